# baseline (speedup 1.0000x reference)
import functools
import math

import jax
import jax.numpy as jnp
from jax import lax
from jax.experimental import pallas as pl
from jax.experimental.pallas import tpu as pltpu
from jax.experimental.pallas import tpu_sc as plsc

EMB_SIZE = 64
SCALE = math.sqrt(EMB_SIZE)

NUM_CORES = 2
NUM_SUBCORES = 16
NUM_WORKERS = NUM_CORES * NUM_SUBCORES
LANES = 16

IDX_ROW = 128
GATHERS_PER_CHUNK = 4
CHUNK = IDX_ROW * GATHERS_PER_CHUNK


def _emb_body(n_chunks, tok_hbm, table_hbm, out_hbm, idx_v, rows_v, sem0, sem1):
  sems = (sem0, sem1)
  ipw = n_chunks * GATHERS_PER_CHUNK
  bpw = n_chunks * CHUNK
  wid = lax.axis_index("s") * NUM_CORES + lax.axis_index("c")
  irow0 = wid * ipw
  row0 = wid * bpw

  pltpu.sync_copy(tok_hbm.at[pl.ds(irow0, ipw), :], idx_v)

  def gather_descs(k, b):
    return [
        pltpu.make_async_copy(
            table_hbm.at[idx_v.at[k * GATHERS_PER_CHUNK + j]],
            rows_v.at[b, pl.ds(j * IDX_ROW, IDX_ROW), :],
            sems[b],
        )
        for j in range(GATHERS_PER_CHUNK)
    ]

  def issue(k, b):
    for d in gather_descs(k, b):
      d.start()

  issue(0, 0)

  @pl.loop(0, n_chunks // 2)
  def _(k2):
    for b in range(2):
      k = k2 * 2 + b

      @pl.when(k + 1 < n_chunks)
      def _():
        issue(k + 1, 1 - b)

      for d in gather_descs(k, b):
        d.wait()

      @pl.loop(0, CHUNK)
      def _(i):
        for jj in range(EMB_SIZE // LANES):
          sl = pl.ds(jj * LANES, LANES)
          rows_v[b, i, sl] = rows_v[b, i, sl] * SCALE

      pltpu.sync_copy(
          rows_v.at[b], out_hbm.at[pl.ds(row0 + k * CHUNK, CHUNK), :]
      )


@functools.partial(jax.jit, static_argnames=("n_chunks",))
def _emb_call(tok2d, table, n_chunks):
  b_total = tok2d.shape[0] * tok2d.shape[1]
  mesh = plsc.VectorSubcoreMesh(
      core_axis_name="c", subcore_axis_name="s", num_cores=NUM_CORES
  )
  return pl.kernel(
      functools.partial(_emb_body, n_chunks),
      out_type=jax.ShapeDtypeStruct((b_total, EMB_SIZE), jnp.float32),
      mesh=mesh,
      scratch_types=[
          pltpu.VMEM((n_chunks * GATHERS_PER_CHUNK, IDX_ROW), jnp.int32),
          pltpu.VMEM((2, CHUNK, EMB_SIZE), jnp.float32),
          pltpu.SemaphoreType.DMA,
          pltpu.SemaphoreType.DMA,
      ],
      compiler_params=pltpu.CompilerParams(use_tc_tiling_on_sc=False),
  )(tok2d, table)


def kernel(tokens, embedding_weight):
  n_tok = tokens.shape[0] * tokens.shape[1]
  assert n_tok % (NUM_WORKERS * CHUNK) == 0
  n_chunks = n_tok // (NUM_WORKERS * CHUNK)
  assert n_chunks % 2 == 0
  tok2d = tokens.astype(jnp.int32).reshape(-1, IDX_ROW)
  out = _emb_call(tok2d, embedding_weight, n_chunks)
  return out.reshape(*tokens.shape, EMB_SIZE)
